# baseline (speedup 1.0000x reference)
import functools

import jax
import jax.numpy as jnp
from jax import lax
from jax.experimental import pallas as pl
from jax.experimental.pallas import tpu as pltpu
from jax.experimental.pallas import tpu_sc as plsc

NUM_FRAME = 8
TOPK = 64
TOP_ATTN = 6
P = 196
D = 768
NUM_LAYERS = 12
NUM_HEADS = 12
SEQ = P + 1
W = 256

SC_CORES = 2
SC_SUBCORES = 16
SC_WORKERS = SC_CORES * SC_SUBCORES


def _topk_body(am_ref, idx_ref, gidx_ref, *, rows):
    nl = NUM_LAYERS - TOP_ATTN
    x = am_ref[...].reshape(rows, nl * NUM_HEADS, W)
    s = jnp.sum(x, axis=1)

    lane = lax.broadcasted_iota(jnp.int32, (rows, W), 1)
    valid = (lane >= 1) & (lane <= P)
    s = jnp.where(valid, s, -jnp.inf)

    lane64 = lax.broadcasted_iota(jnp.int32, (rows, TOPK), 1)

    def step(k, carry):
        sc, idx = carry
        m = jnp.max(sc, axis=1, keepdims=True)
        hit = sc == m
        lk = jnp.min(jnp.where(hit, lane, W), axis=1, keepdims=True)
        idx = jnp.where(lane64 == k, lk - 1, idx)
        sc = jnp.where(lane == lk, -jnp.inf, sc)
        return sc, idx

    idx0 = jnp.zeros((rows, TOPK), jnp.int32)
    _, idx = lax.fori_loop(0, TOPK, step, (s, idx0))

    idx_ref[...] = idx
    row = lax.broadcasted_iota(jnp.int32, (rows, TOPK), 0)
    gidx_ref[...] = idx + row * P


@functools.lru_cache(maxsize=None)
def _make_sc_gather(n_rows, d):
    per_w = n_rows // SC_WORKERS
    mesh = plsc.VectorSubcoreMesh(core_axis_name="c", subcore_axis_name="s")

    @functools.partial(
        pl.kernel,
        mesh=mesh,
        out_type=jax.ShapeDtypeStruct((n_rows, d), jnp.float32),
        scratch_types=[
            pltpu.VMEM((per_w,), jnp.int32),
            pltpu.VMEM((per_w, d), jnp.float32),
            pltpu.SemaphoreType.DMA,
        ],
    )
    def sc_gather(table_hbm, idx_hbm, out_hbm, idx_v, rows_v, sem):
        wid = lax.axis_index("s") * SC_CORES + lax.axis_index("c")
        base = wid * per_w
        pltpu.sync_copy(idx_hbm.at[pl.ds(base, per_w)], idx_v)
        pltpu.async_copy(table_hbm.at[idx_v], rows_v, sem).wait()
        pltpu.sync_copy(rows_v, out_hbm.at[pl.ds(base, per_w)])

    return sc_gather


def kernel(tokens, attn_maps):
    B = tokens.shape[0]
    rows = B * NUM_FRAME

    am_view = attn_maps.reshape(rows, NUM_LAYERS, NUM_HEADS, SEQ * SEQ)

    idx, gidx = pl.pallas_call(
        functools.partial(_topk_body, rows=rows),
        grid=(1,),
        in_specs=[
            pl.BlockSpec(
                (rows, NUM_LAYERS - TOP_ATTN, NUM_HEADS, W),
                lambda i: (0, 1, 0, 0),
            )
        ],
        out_specs=[
            pl.BlockSpec((rows, TOPK), lambda i: (0, 0)),
            pl.BlockSpec((rows, TOPK), lambda i: (0, 0)),
        ],
        out_shape=[
            jax.ShapeDtypeStruct((rows, TOPK), jnp.int32),
            jax.ShapeDtypeStruct((rows, TOPK), jnp.int32),
        ],
    )(am_view)

    n_rows = rows * TOPK
    gather = _make_sc_gather(n_rows, D)
    out = gather(tokens.reshape(B * NUM_FRAME * P, D), gidx.reshape(n_rows))

    return out.reshape(B, NUM_FRAME * TOPK, D), idx.reshape(B, NUM_FRAME, TOPK)


# speedup vs baseline: 2.3659x; 2.3659x over previous
import functools

import jax
import jax.numpy as jnp
from jax import lax
from jax.experimental import pallas as pl
from jax.experimental.pallas import tpu as pltpu
from jax.experimental.pallas import tpu_sc as plsc

NUM_FRAME = 8
TOPK = 64
TOP_ATTN = 6
P = 196
D = 768
NUM_LAYERS = 12
NUM_HEADS = 12
SEQ = P + 1
W = SEQ

SC_CORES = 2
SC_SUBCORES = 16
SC_WORKERS = SC_CORES * SC_SUBCORES


def _topk_body(am_ref, idx_ref, gidx_ref, *, batch):
    rows = batch * NUM_FRAME
    x = am_ref[:, :, :, :, 0, :]
    s = jnp.sum(jnp.sum(x, axis=3), axis=2)
    s = s.reshape(rows, W)

    lane = lax.broadcasted_iota(jnp.int32, (rows, W), 1)
    s = jnp.where(lane >= 1, s, -jnp.inf)

    lane64 = lax.broadcasted_iota(jnp.int32, (rows, TOPK), 1)

    def step(k, carry):
        sc, idx = carry
        m = jnp.max(sc, axis=1, keepdims=True)
        hit = sc == m
        lk = jnp.min(jnp.where(hit, lane, W), axis=1, keepdims=True)
        idx = jnp.where(lane64 == k, lk - 1, idx)
        sc = jnp.where(lane == lk, -jnp.inf, sc)
        return sc, idx

    idx0 = jnp.zeros((rows, TOPK), jnp.int32)
    _, idx = lax.fori_loop(0, TOPK, step, (s, idx0))

    idx_ref[...] = idx
    row = lax.broadcasted_iota(jnp.int32, (rows, TOPK), 0)
    gidx_ref[...] = idx + row * P


@functools.lru_cache(maxsize=None)
def _make_sc_gather(n_rows, d):
    per_w = n_rows // SC_WORKERS
    mesh = plsc.VectorSubcoreMesh(core_axis_name="c", subcore_axis_name="s")

    @functools.partial(
        pl.kernel,
        mesh=mesh,
        out_type=jax.ShapeDtypeStruct((n_rows, d), jnp.float32),
        scratch_types=[
            pltpu.VMEM((per_w,), jnp.int32),
            pltpu.VMEM((per_w, d), jnp.float32),
            pltpu.SemaphoreType.DMA,
        ],
    )
    def sc_gather(table_hbm, idx_hbm, out_hbm, idx_v, rows_v, sem):
        wid = lax.axis_index("s") * SC_CORES + lax.axis_index("c")
        base = wid * per_w
        pltpu.sync_copy(idx_hbm.at[pl.ds(base, per_w)], idx_v)
        pltpu.async_copy(table_hbm.at[idx_v], rows_v, sem).wait()
        pltpu.sync_copy(rows_v, out_hbm.at[pl.ds(base, per_w)])

    return sc_gather


def kernel(tokens, attn_maps):
    B = tokens.shape[0]
    rows = B * NUM_FRAME

    idx, gidx = pl.pallas_call(
        functools.partial(_topk_body, batch=B),
        grid=(1,),
        in_specs=[
            pl.BlockSpec(
                (B, NUM_FRAME, NUM_LAYERS - TOP_ATTN, NUM_HEADS, 8, W),
                lambda i: (0, 0, 1, 0, 0, 0),
            )
        ],
        out_specs=[
            pl.BlockSpec((rows, TOPK), lambda i: (0, 0)),
            pl.BlockSpec((rows, TOPK), lambda i: (0, 0)),
        ],
        out_shape=[
            jax.ShapeDtypeStruct((rows, TOPK), jnp.int32),
            jax.ShapeDtypeStruct((rows, TOPK), jnp.int32),
        ],
    )(attn_maps)

    n_rows = rows * TOPK
    gather = _make_sc_gather(n_rows, D)
    out = gather(tokens.reshape(B * NUM_FRAME * P, D), gidx.reshape(n_rows))

    return out.reshape(B, NUM_FRAME * TOPK, D), idx.reshape(B, NUM_FRAME, TOPK)
